# Initial kernel scaffold; baseline (speedup 1.0000x reference)
import jax
import jax.numpy as jnp
from jax.experimental import pallas as pl


def kernel(input_labels, out_labels, num_sampled, in_embed_weight, out_embed_weight):
    raise NotImplementedError("write your pallas kernel here")


# speedup vs baseline: 1.2638x; 1.2638x over previous
import functools

import jax
import jax.numpy as jnp
from jax import lax
from jax.experimental import pallas as pl
from jax.experimental.pallas import tpu as pltpu
from jax.experimental.pallas import tpu_sc as plsc

_TYPE_OFFSET = [0, 500000, 1000000]
_EDGE_TYPES = [[0, 1, 0], [1, 0, 0]]
_NS = 5
_D = 64
_NW = 32
_G = 16

_LN2 = 0.6931471805599453


def _log_sig(x):
    y = jnp.clip(x, -6.0, 6.0)
    z = 1.0 + jnp.exp(-y)
    b = lax.bitcast_convert_type(z, jnp.int32)
    e = jnp.right_shift(b, 23) - 127
    mb = jnp.bitwise_or(jnp.bitwise_and(b, 0x007FFFFF), 0x3F800000)
    m = lax.bitcast_convert_type(mb, jnp.float32)
    big = m > 1.5
    m = jnp.where(big, m * 0.5, m)
    e = (e + big.astype(jnp.int32)).astype(jnp.float32)
    t = (m - 1.0) / (m + 1.0)
    t2 = t * t
    p = 2.0 * t * (1.0 + t2 * (1.0 / 3.0 + t2 * (0.2 + t2 * (1.0 / 7.0 + t2 * (1.0 / 9.0)))))
    return -(e * _LN2 + p)


def _make_sc_kernel(chunk, n_groups):
    mesh = plsc.VectorSubcoreMesh(core_axis_name="c", subcore_axis_name="s")
    scratch = (
        [pltpu.VMEM((chunk,), jnp.int32) for _ in range(2)]
        + [pltpu.VMEM((chunk * _NS,), jnp.int32) for _ in range(2)]
        + [pltpu.VMEM((_G, _D), jnp.float32) for _ in range(8)]
        + [pltpu.VMEM((_G * _NS, _D), jnp.float32) for _ in range(8)]
        + [pltpu.VMEM((6 * 16,), jnp.float32),
           pltpu.SemaphoreType.DMA, pltpu.SemaphoreType.DMA]
    )

    @functools.partial(
        pl.kernel, mesh=mesh,
        out_type=jax.ShapeDtypeStruct((_NW * 96,), jnp.float32),
        scratch_types=scratch,
        compiler_params=pltpu.CompilerParams(
            needs_layout_passes=False, use_tc_tiling_on_sc=False),
    )
    def run(pos_u, pos_v, un, vn, tin, tout, out,
            pu_v, pv_v, un_v, vn_v,
            ai0, ao0, vi0, vo0, ai1, ao1, vi1, vo1,
            uni0, uno0, vni0, vno0, uni1, uno1, vni1, vno1,
            stage, sem_a, sem_b):
        wid = lax.axis_index("s") * 2 + lax.axis_index("c")
        base = pl.multiple_of(wid * chunk, chunk)
        base5 = pl.multiple_of(wid * (chunk * _NS), chunk * _NS)
        pltpu.sync_copy(pos_u.at[pl.ds(base, chunk)], pu_v)
        pltpu.sync_copy(pos_v.at[pl.ds(base, chunk)], pv_v)
        pltpu.sync_copy(un.at[pl.ds(base5, chunk * _NS)], un_v)
        pltpu.sync_copy(vn.at[pl.ds(base5, chunk * _NS)], vn_v)

        bufs0 = (ai0, ao0, vi0, vo0, uni0, uno0, vni0, vno0)
        bufs1 = (ai1, ao1, vi1, vo1, uni1, uno1, vni1, vno1)

        def handles(g, bufs, sem):
            o = pl.multiple_of(g * _G, _G)
            o5 = pl.multiple_of(g * (_G * _NS), _G * _NS)
            ai, ao, vi, vo, uni, uno, vni, vno = bufs
            ipu = pu_v.at[pl.ds(o, _G)]
            ipv = pv_v.at[pl.ds(o, _G)]
            iun = un_v.at[pl.ds(o5, _G * _NS)]
            ivn = vn_v.at[pl.ds(o5, _G * _NS)]
            return [
                pltpu.make_async_copy(tin.at[ipu], ai, sem),
                pltpu.make_async_copy(tout.at[ipu], ao, sem),
                pltpu.make_async_copy(tin.at[ipv], vi, sem),
                pltpu.make_async_copy(tout.at[ipv], vo, sem),
                pltpu.make_async_copy(tin.at[iun], uni, sem),
                pltpu.make_async_copy(tout.at[iun], uno, sem),
                pltpu.make_async_copy(tin.at[ivn], vni, sem),
                pltpu.make_async_copy(tout.at[ivn], vno, sem),
            ]

        def fire(g, bufs, sem):
            for h in handles(g, bufs, sem):
                h.start()

        def drain(g, bufs, sem):
            for h in handles(g, bufs, sem):
                h.wait()

        iota = lax.iota(jnp.int32, 16)
        rn = [iota * _NS + k for k in range(_NS)]
        zeros = jnp.zeros((16,), jnp.float32)
        zcol = jnp.zeros((16,), jnp.int32)

        def compute(bufs, accs):
            ai, ao, vi, vo, uni, uno, vni, vno = bufs

            def dbody(_, c):
                dots = list(c[:22])
                dc = c[22]
                a_i = plsc.load_gather(ai, [iota, dc])
                a_o = plsc.load_gather(ao, [iota, dc])
                v_i = plsc.load_gather(vi, [iota, dc])
                v_o = plsc.load_gather(vo, [iota, dc])
                dots[0] = dots[0] + a_i * v_i
                dots[1] = dots[1] + a_o * v_o
                for k in range(_NS):
                    u_i = plsc.load_gather(uni, [rn[k], dc])
                    u_o = plsc.load_gather(uno, [rn[k], dc])
                    w_i = plsc.load_gather(vni, [rn[k], dc])
                    w_o = plsc.load_gather(vno, [rn[k], dc])
                    dots[2 + k] = dots[2 + k] + u_i * v_i
                    dots[7 + k] = dots[7 + k] + u_o * v_o
                    dots[12 + k] = dots[12 + k] + w_i * a_i
                    dots[17 + k] = dots[17 + k] + w_o * a_o
                return (*dots, dc + 1)

            dots = lax.fori_loop(0, _D, dbody, (*((zeros,) * 22), zcol))
            acc_pos = accs[0] + _log_sig(dots[0]) + _log_sig(dots[1])
            news = [acc_pos]
            for k in range(_NS):
                news.append(accs[1 + k]
                            + _log_sig(-dots[2 + k]) + _log_sig(-dots[7 + k])
                            + _log_sig(-dots[12 + k]) + _log_sig(-dots[17 + k]))
            return tuple(news)

        fire(0, bufs0, sem_a)
        fire(1, bufs1, sem_b)

        def gbody(i, accs):
            g0 = 2 * i
            drain(g0, bufs0, sem_a)
            accs = compute(bufs0, accs)

            @pl.when(i < n_groups // 2 - 1)
            def _():
                fire(g0 + 2, bufs0, sem_a)

            drain(g0 + 1, bufs1, sem_b)
            accs = compute(bufs1, accs)

            @pl.when(i < n_groups // 2 - 1)
            def _():
                fire(g0 + 3, bufs1, sem_b)

            return accs

        accs = lax.fori_loop(0, n_groups // 2, gbody, (zeros,) * 6)
        for r in range(6):
            stage[pl.ds(r * 16, 16)] = accs[r]
        pltpu.sync_copy(stage, out.at[pl.ds(pl.multiple_of(wid * 96, 96), 96)])

    return run


def _noise_idx(key, n, ns, lo, hi):
    span = float(hi - lo - 1)
    return jnp.floor(jax.random.uniform(key, (n, ns)) * span).astype(jnp.int32) + lo


def kernel(input_labels, out_labels, num_sampled, in_embed_weight, out_embed_weight):
    B, wp1 = out_labels.shape
    W = wp1 - 1
    BW = B * W
    types = input_labels[:, 0]
    ids = input_labels[:, 1]
    j = jnp.arange(BW)

    pu, pv, un, vn = [], [], [], []
    n0 = None
    for tp in range(len(_EDGE_TYPES)):
        tu, tv, _ = _EDGE_TYPES[tp]
        sel = types == tp
        idxa = jnp.nonzero(sel, size=B, fill_value=0)[0]
        n = jnp.sum(sel.astype(jnp.int32))
        pu.append(ids[idxa[j % jnp.maximum(n, 1)]])
        pv.append(out_labels[idxa[j // W], 1 + j % W])
        un.append(_noise_idx(jax.random.fold_in(jax.random.key(1), tp), BW, _NS,
                             _TYPE_OFFSET[tu], _TYPE_OFFSET[tu + 1]))
        vn.append(_noise_idx(jax.random.fold_in(jax.random.key(2), tp), BW, _NS,
                             _TYPE_OFFSET[tv], _TYPE_OFFSET[tv + 1]))
        if tp == 0:
            n0 = n

    is0 = j < n0 * W
    jm = jnp.where(is0, j, j - n0 * W)
    pos_u = jnp.where(is0, pu[0][jm], pu[1][jm]).astype(jnp.int32)
    pos_v = jnp.where(is0, pv[0][jm], pv[1][jm]).astype(jnp.int32)
    unc = jnp.where(is0[:, None], un[0][jm], un[1][jm]).reshape(-1)
    vnc = jnp.where(is0[:, None], vn[0][jm], vn[1][jm]).reshape(-1)

    chunk = BW // _NW
    run = _make_sc_kernel(chunk, chunk // _G)
    parts = run(pos_u, pos_v, unc, vnc, in_embed_weight, out_embed_weight)
    parts = parts.reshape(_NW, 6, 16)
    pos_sum = parts[:, 0, :].sum()
    s = parts[:, 1:, :].sum(axis=(0, 2))
    colmask = jnp.arange(_NS) < num_sampled
    total = pos_sum + jnp.where(colmask, s, 0.0).sum() * 0.5
    return -total / BW


# speedup vs baseline: 1.6609x; 1.3142x over previous
import functools

import jax
import jax.numpy as jnp
from jax import lax
from jax.experimental import pallas as pl
from jax.experimental.pallas import tpu as pltpu
from jax.experimental.pallas import tpu_sc as plsc

_TYPE_OFFSET = [0, 500000, 1000000]
_EDGE_TYPES = [[0, 1, 0], [1, 0, 0]]
_NS = 5
_D = 64
_NW = 32
_G = 16

_LN2 = 0.6931471805599453


def _log_sig(x):
    y = jnp.clip(x, -6.0, 6.0)
    z = 1.0 + jnp.exp(-y)
    b = lax.bitcast_convert_type(z, jnp.int32)
    e = jnp.right_shift(b, 23) - 127
    mb = jnp.bitwise_or(jnp.bitwise_and(b, 0x007FFFFF), 0x3F800000)
    m = lax.bitcast_convert_type(mb, jnp.float32)
    big = m > 1.5
    m = jnp.where(big, m * 0.5, m)
    e = (e + big.astype(jnp.int32)).astype(jnp.float32)
    t = (m - 1.0) / (m + 1.0)
    t2 = t * t
    p = 2.0 * t * (1.0 + t2 * (1.0 / 3.0 + t2 * (0.2 + t2 * (1.0 / 7.0 + t2 * (1.0 / 9.0)))))
    return -(e * _LN2 + p)


def _make_sc_kernel(chunk, n_groups):
    mesh = plsc.VectorSubcoreMesh(core_axis_name="c", subcore_axis_name="s")
    scratch = (
        [pltpu.VMEM((chunk,), jnp.int32) for _ in range(2)]
        + [pltpu.VMEM((chunk * _NS,), jnp.int32) for _ in range(2)]
        + [pltpu.VMEM((_G, _D), jnp.float32) for _ in range(8)]
        + [pltpu.VMEM((_G * _NS, _D), jnp.float32) for _ in range(8)]
        + [pltpu.VMEM((6 * 16,), jnp.float32),
           pltpu.SemaphoreType.DMA, pltpu.SemaphoreType.DMA]
    )

    @functools.partial(
        pl.kernel, mesh=mesh,
        out_type=jax.ShapeDtypeStruct((_NW * 96,), jnp.float32),
        scratch_types=scratch,
        compiler_params=pltpu.CompilerParams(
            needs_layout_passes=False, use_tc_tiling_on_sc=False),
    )
    def run(pos_u, pos_v, un, vn, tin, tout, out,
            pu_v, pv_v, un_v, vn_v,
            ai0, ao0, vi0, vo0, ai1, ao1, vi1, vo1,
            uni0, uno0, vni0, vno0, uni1, uno1, vni1, vno1,
            stage, sem_a, sem_b):
        wid = lax.axis_index("s") * 2 + lax.axis_index("c")
        base = pl.multiple_of(wid * chunk, chunk)
        base5 = pl.multiple_of(wid * (chunk * _NS), chunk * _NS)
        pltpu.sync_copy(pos_u.at[pl.ds(base, chunk)], pu_v)
        pltpu.sync_copy(pos_v.at[pl.ds(base, chunk)], pv_v)
        pltpu.sync_copy(un.at[pl.ds(base5, chunk * _NS)], un_v)
        pltpu.sync_copy(vn.at[pl.ds(base5, chunk * _NS)], vn_v)

        bufs0 = (ai0, ao0, vi0, vo0, uni0, uno0, vni0, vno0)
        bufs1 = (ai1, ao1, vi1, vo1, uni1, uno1, vni1, vno1)

        def handles(g, bufs, sem):
            o = pl.multiple_of(g * _G, _G)
            o5 = pl.multiple_of(g * (_G * _NS), _G * _NS)
            ai, ao, vi, vo, uni, uno, vni, vno = bufs
            ipu = pu_v.at[pl.ds(o, _G)]
            ipv = pv_v.at[pl.ds(o, _G)]
            iun = un_v.at[pl.ds(o5, _G * _NS)]
            ivn = vn_v.at[pl.ds(o5, _G * _NS)]
            return [
                pltpu.make_async_copy(tin.at[ipu], ai, sem),
                pltpu.make_async_copy(tout.at[ipu], ao, sem),
                pltpu.make_async_copy(tin.at[ipv], vi, sem),
                pltpu.make_async_copy(tout.at[ipv], vo, sem),
                pltpu.make_async_copy(tin.at[iun], uni, sem),
                pltpu.make_async_copy(tout.at[iun], uno, sem),
                pltpu.make_async_copy(tin.at[ivn], vni, sem),
                pltpu.make_async_copy(tout.at[ivn], vno, sem),
            ]

        def fire(g, bufs, sem):
            for h in handles(g, bufs, sem):
                h.start()

        def drain(g, bufs, sem):
            for h in handles(g, bufs, sem):
                h.wait()

        iota = lax.iota(jnp.int32, 16)
        rn = [iota * _NS + k for k in range(_NS)]
        zeros = jnp.zeros((16,), jnp.float32)
        zcol = jnp.zeros((16,), jnp.int32)

        def halfdots(a, v, un, vn):
            def dbody(_, c):
                dots = list(c[:11])
                dc = c[11]
                a_x = plsc.load_gather(a, [iota, dc])
                v_x = plsc.load_gather(v, [iota, dc])
                dots[0] = dots[0] + a_x * v_x
                for k in range(_NS):
                    u_x = plsc.load_gather(un, [rn[k], dc])
                    w_x = plsc.load_gather(vn, [rn[k], dc])
                    dots[1 + k] = dots[1 + k] + u_x * v_x
                    dots[6 + k] = dots[6 + k] + w_x * a_x
                return (*dots, dc + 1)

            out = lax.fori_loop(0, _D, dbody, (*((zeros,) * 11), zcol),
                                unroll=2)
            return out[:11]

        def compute(bufs, accs):
            ai, ao, vi, vo, uni, uno, vni, vno = bufs
            din = halfdots(ai, vi, uni, vni)
            dout = halfdots(ao, vo, uno, vno)
            acc_pos = accs[0] + _log_sig(din[0]) + _log_sig(dout[0])
            news = [acc_pos]
            for k in range(_NS):
                news.append(accs[1 + k]
                            + _log_sig(-din[1 + k]) + _log_sig(-dout[1 + k])
                            + _log_sig(-din[6 + k]) + _log_sig(-dout[6 + k]))
            return tuple(news)

        fire(0, bufs0, sem_a)
        fire(1, bufs1, sem_b)

        def gbody(i, accs):
            g0 = 2 * i
            drain(g0, bufs0, sem_a)
            accs = compute(bufs0, accs)

            @pl.when(i < n_groups // 2 - 1)
            def _():
                fire(g0 + 2, bufs0, sem_a)

            drain(g0 + 1, bufs1, sem_b)
            accs = compute(bufs1, accs)

            @pl.when(i < n_groups // 2 - 1)
            def _():
                fire(g0 + 3, bufs1, sem_b)

            return accs

        accs = lax.fori_loop(0, n_groups // 2, gbody, (zeros,) * 6)
        for r in range(6):
            stage[pl.ds(r * 16, 16)] = accs[r]
        pltpu.sync_copy(stage, out.at[pl.ds(pl.multiple_of(wid * 96, 96), 96)])

    return run


def _noise_idx(key, n, ns, lo, hi):
    span = float(hi - lo - 1)
    return jnp.floor(jax.random.uniform(key, (n, ns)) * span).astype(jnp.int32) + lo


def kernel(input_labels, out_labels, num_sampled, in_embed_weight, out_embed_weight):
    B, wp1 = out_labels.shape
    W = wp1 - 1
    BW = B * W
    types = input_labels[:, 0]
    ids = input_labels[:, 1]
    j = jnp.arange(BW)

    pu, pv, un, vn = [], [], [], []
    n0 = None
    for tp in range(len(_EDGE_TYPES)):
        tu, tv, _ = _EDGE_TYPES[tp]
        sel = types == tp
        idxa = jnp.nonzero(sel, size=B, fill_value=0)[0]
        n = jnp.sum(sel.astype(jnp.int32))
        pu.append(ids[idxa[j % jnp.maximum(n, 1)]])
        pv.append(out_labels[idxa[j // W], 1 + j % W])
        un.append(_noise_idx(jax.random.fold_in(jax.random.key(1), tp), BW, _NS,
                             _TYPE_OFFSET[tu], _TYPE_OFFSET[tu + 1]))
        vn.append(_noise_idx(jax.random.fold_in(jax.random.key(2), tp), BW, _NS,
                             _TYPE_OFFSET[tv], _TYPE_OFFSET[tv + 1]))
        if tp == 0:
            n0 = n

    is0 = j < n0 * W
    shift = n0 * W
    pos_u = jnp.where(is0, pu[0], jnp.roll(pu[1], shift)).astype(jnp.int32)
    pos_v = jnp.where(is0, pv[0], jnp.roll(pv[1], shift)).astype(jnp.int32)
    unc = jnp.where(is0[:, None], un[0], jnp.roll(un[1], shift, axis=0)).reshape(-1)
    vnc = jnp.where(is0[:, None], vn[0], jnp.roll(vn[1], shift, axis=0)).reshape(-1)

    chunk = BW // _NW
    run = _make_sc_kernel(chunk, chunk // _G)
    parts = run(pos_u, pos_v, unc, vnc, in_embed_weight, out_embed_weight)
    parts = parts.reshape(_NW, 6, 16)
    pos_sum = parts[:, 0, :].sum()
    s = parts[:, 1:, :].sum(axis=(0, 2))
    colmask = jnp.arange(_NS) < num_sampled
    total = pos_sum + jnp.where(colmask, s, 0.0).sum() * 0.5
    return -total / BW


# speedup vs baseline: 3.2678x; 1.9675x over previous
import functools

import jax
import jax.numpy as jnp
from jax import lax
from jax.experimental import pallas as pl
from jax.experimental.pallas import tpu as pltpu
from jax.experimental.pallas import tpu_sc as plsc

_TYPE_OFFSET = [0, 500000, 1000000]
_EDGE_TYPES = [[0, 1, 0], [1, 0, 0]]
_NS = 5
_D = 64
_NW = 32
_G = 16

_LN2 = 0.6931471805599453


def _log_sig(x):
    y = jnp.clip(x, -6.0, 6.0)
    z = 1.0 + jnp.exp(-y)
    b = lax.bitcast_convert_type(z, jnp.int32)
    e = jnp.right_shift(b, 23) - 127
    mb = jnp.bitwise_or(jnp.bitwise_and(b, 0x007FFFFF), 0x3F800000)
    m = lax.bitcast_convert_type(mb, jnp.float32)
    big = m > 1.5
    m = jnp.where(big, m * 0.5, m)
    e = (e + big.astype(jnp.int32)).astype(jnp.float32)
    t = (m - 1.0) / (m + 1.0)
    t2 = t * t
    p = 2.0 * t * (1.0 + t2 * (1.0 / 3.0 + t2 * (0.2 + t2 * (1.0 / 7.0 + t2 * (1.0 / 9.0)))))
    return -(e * _LN2 + p)


def _make_sc_kernel(chunk, n_groups):
    mesh = plsc.VectorSubcoreMesh(core_axis_name="c", subcore_axis_name="s")
    scratch = (
        [pltpu.VMEM((chunk,), jnp.int32) for _ in range(2)]
        + [pltpu.VMEM((chunk * _NS,), jnp.int32) for _ in range(2)]
        + [pltpu.VMEM((_G, _D), jnp.float32) for _ in range(8)]
        + [pltpu.VMEM((_G * _NS, _D), jnp.float32) for _ in range(8)]
        + [pltpu.VMEM((6 * 16,), jnp.float32),
           pltpu.SemaphoreType.DMA, pltpu.SemaphoreType.DMA]
    )

    @functools.partial(
        pl.kernel, mesh=mesh,
        out_type=jax.ShapeDtypeStruct((_NW * 96,), jnp.float32),
        scratch_types=scratch,
        compiler_params=pltpu.CompilerParams(
            needs_layout_passes=False, use_tc_tiling_on_sc=False),
    )
    def run(pos_u, pos_v, un, vn, tin, tout, out,
            pu_v, pv_v, un_v, vn_v,
            ai0, ao0, vi0, vo0, ai1, ao1, vi1, vo1,
            uni0, uno0, vni0, vno0, uni1, uno1, vni1, vno1,
            stage, sem_a, sem_b):
        wid = lax.axis_index("s") * 2 + lax.axis_index("c")
        base = pl.multiple_of(wid * chunk, chunk)
        base5 = pl.multiple_of(wid * (chunk * _NS), chunk * _NS)
        pltpu.sync_copy(pos_u.at[pl.ds(base, chunk)], pu_v)
        pltpu.sync_copy(pos_v.at[pl.ds(base, chunk)], pv_v)
        pltpu.sync_copy(un.at[pl.ds(base5, chunk * _NS)], un_v)
        pltpu.sync_copy(vn.at[pl.ds(base5, chunk * _NS)], vn_v)

        bufs0 = (ai0, ao0, vi0, vo0, uni0, uno0, vni0, vno0)
        bufs1 = (ai1, ao1, vi1, vo1, uni1, uno1, vni1, vno1)

        def handles(g, bufs, sem):
            o = pl.multiple_of(g * _G, _G)
            o5 = pl.multiple_of(g * (_G * _NS), _G * _NS)
            ai, ao, vi, vo, uni, uno, vni, vno = bufs
            ipu = pu_v.at[pl.ds(o, _G)]
            ipv = pv_v.at[pl.ds(o, _G)]
            iun = un_v.at[pl.ds(o5, _G * _NS)]
            ivn = vn_v.at[pl.ds(o5, _G * _NS)]
            return [
                pltpu.make_async_copy(tin.at[ipu], ai, sem),
                pltpu.make_async_copy(tout.at[ipu], ao, sem),
                pltpu.make_async_copy(tin.at[ipv], vi, sem),
                pltpu.make_async_copy(tout.at[ipv], vo, sem),
                pltpu.make_async_copy(tin.at[iun], uni, sem),
                pltpu.make_async_copy(tout.at[iun], uno, sem),
                pltpu.make_async_copy(tin.at[ivn], vni, sem),
                pltpu.make_async_copy(tout.at[ivn], vno, sem),
            ]

        def fire(g, bufs, sem):
            for h in handles(g, bufs, sem):
                h.start()

        def drain(g, bufs, sem):
            for h in handles(g, bufs, sem):
                h.wait()

        iota = lax.iota(jnp.int32, 16)
        rn = [iota * _NS + k for k in range(_NS)]
        zeros = jnp.zeros((16,), jnp.float32)
        zcol = jnp.zeros((16,), jnp.int32)

        def halfdots(a, v, un, vn):
            def dbody(_, c):
                dots = list(c[:11])
                dc = c[11]
                a_x = plsc.load_gather(a, [iota, dc])
                v_x = plsc.load_gather(v, [iota, dc])
                dots[0] = dots[0] + a_x * v_x
                for k in range(_NS):
                    u_x = plsc.load_gather(un, [rn[k], dc])
                    w_x = plsc.load_gather(vn, [rn[k], dc])
                    dots[1 + k] = dots[1 + k] + u_x * v_x
                    dots[6 + k] = dots[6 + k] + w_x * a_x
                return (*dots, (dc + 1) & (_D - 1))

            out = lax.fori_loop(0, _D, dbody, (*((zeros,) * 11), iota),
                                unroll=2)
            return out[:11]

        def compute(bufs, accs):
            ai, ao, vi, vo, uni, uno, vni, vno = bufs
            din = halfdots(ai, vi, uni, vni)
            dout = halfdots(ao, vo, uno, vno)
            acc_pos = accs[0] + _log_sig(din[0]) + _log_sig(dout[0])
            news = [acc_pos]
            for k in range(_NS):
                news.append(accs[1 + k]
                            + _log_sig(-din[1 + k]) + _log_sig(-dout[1 + k])
                            + _log_sig(-din[6 + k]) + _log_sig(-dout[6 + k]))
            return tuple(news)

        fire(0, bufs0, sem_a)
        fire(1, bufs1, sem_b)

        def gbody(i, accs):
            g0 = 2 * i
            drain(g0, bufs0, sem_a)
            accs = compute(bufs0, accs)

            @pl.when(i < n_groups // 2 - 1)
            def _():
                fire(g0 + 2, bufs0, sem_a)

            drain(g0 + 1, bufs1, sem_b)
            accs = compute(bufs1, accs)

            @pl.when(i < n_groups // 2 - 1)
            def _():
                fire(g0 + 3, bufs1, sem_b)

            return accs

        accs = lax.fori_loop(0, n_groups // 2, gbody, (zeros,) * 6)
        for r in range(6):
            stage[pl.ds(r * 16, 16)] = accs[r]
        pltpu.sync_copy(stage, out.at[pl.ds(pl.multiple_of(wid * 96, 96), 96)])

    return run


def _noise_idx(key, n, ns, lo, hi):
    span = float(hi - lo - 1)
    return jnp.floor(jax.random.uniform(key, (n * ns,)) * span).astype(jnp.int32) + lo


def kernel(input_labels, out_labels, num_sampled, in_embed_weight, out_embed_weight):
    B, wp1 = out_labels.shape
    W = wp1 - 1
    BW = B * W
    types = input_labels[:, 0]
    ids = input_labels[:, 1]
    olf = out_labels.reshape(-1)
    j = jnp.arange(BW)
    jW = j // W
    jR = j % W

    pu, pv, un, vn = [], [], [], []
    n0 = None
    for tp in range(len(_EDGE_TYPES)):
        tu, tv, _ = _EDGE_TYPES[tp]
        sel = types == tp
        idxa = jnp.nonzero(sel, size=B, fill_value=0)[0]
        n = jnp.sum(sel.astype(jnp.int32))
        c_tp = lax.optimization_barrier(ids[idxa])
        pu.append(c_tp[j % jnp.maximum(n, 1)])
        r2 = lax.optimization_barrier(idxa[jW])
        pv.append(olf[r2 * wp1 + 1 + jR])
        un.append(_noise_idx(jax.random.fold_in(jax.random.key(1), tp), BW, _NS,
                             _TYPE_OFFSET[tu], _TYPE_OFFSET[tu + 1]))
        vn.append(_noise_idx(jax.random.fold_in(jax.random.key(2), tp), BW, _NS,
                             _TYPE_OFFSET[tv], _TYPE_OFFSET[tv + 1]))
        if tp == 0:
            n0 = n

    shift = n0 * W
    is0 = j < shift
    is0f = jnp.arange(BW * _NS) < shift * _NS
    pos_u = jnp.where(is0, pu[0], jnp.roll(pu[1], shift)).astype(jnp.int32)
    pos_v = jnp.where(is0, pv[0], jnp.roll(pv[1], shift)).astype(jnp.int32)
    unc = jnp.where(is0f, un[0], jnp.roll(un[1], shift * _NS))
    vnc = jnp.where(is0f, vn[0], jnp.roll(vn[1], shift * _NS))

    chunk = BW // _NW
    run = _make_sc_kernel(chunk, chunk // _G)
    parts = run(pos_u, pos_v, unc, vnc, in_embed_weight, out_embed_weight)
    parts = parts.reshape(_NW, 6, 16)
    pos_sum = parts[:, 0, :].sum()
    s = parts[:, 1:, :].sum(axis=(0, 2))
    colmask = jnp.arange(_NS) < num_sampled
    total = pos_sum + jnp.where(colmask, s, 0.0).sum() * 0.5
    return -total / BW


# speedup vs baseline: 3.3123x; 1.0136x over previous
import functools

import jax
import jax.numpy as jnp
from jax import lax
from jax.experimental import pallas as pl
from jax.experimental.pallas import tpu as pltpu
from jax.experimental.pallas import tpu_sc as plsc

_TYPE_OFFSET = [0, 500000, 1000000]
_EDGE_TYPES = [[0, 1, 0], [1, 0, 0]]
_NS = 5
_D = 64
_NW = 32
_G = 16

_LN2 = 0.6931471805599453


def _log_sig(x):
    y = jnp.clip(x, -6.0, 6.0)
    z = 1.0 + jnp.exp(-y)
    b = lax.bitcast_convert_type(z, jnp.int32)
    e = jnp.right_shift(b, 23) - 127
    mb = jnp.bitwise_or(jnp.bitwise_and(b, 0x007FFFFF), 0x3F800000)
    m = lax.bitcast_convert_type(mb, jnp.float32)
    big = m > 1.5
    m = jnp.where(big, m * 0.5, m)
    e = (e + big.astype(jnp.int32)).astype(jnp.float32)
    t = (m - 1.0) / (m + 1.0)
    t2 = t * t
    p = 2.0 * t * (1.0 + t2 * (1.0 / 3.0 + t2 * (0.2 + t2 * (1.0 / 7.0 + t2 * (1.0 / 9.0)))))
    return -(e * _LN2 + p)


def _make_sc_kernel(chunk, n_groups):
    mesh = plsc.VectorSubcoreMesh(core_axis_name="c", subcore_axis_name="s")
    scratch = (
        [pltpu.VMEM((chunk,), jnp.int32) for _ in range(2)]
        + [pltpu.VMEM((chunk * _NS,), jnp.int32) for _ in range(2)]
        + [pltpu.VMEM((_G, _D), jnp.float32) for _ in range(8)]
        + [pltpu.VMEM((_G * _NS, _D), jnp.float32) for _ in range(8)]
        + [pltpu.VMEM((6 * 16,), jnp.float32),
           pltpu.SemaphoreType.DMA, pltpu.SemaphoreType.DMA]
    )

    @functools.partial(
        pl.kernel, mesh=mesh,
        out_type=jax.ShapeDtypeStruct((_NW * 96,), jnp.float32),
        scratch_types=scratch,
        compiler_params=pltpu.CompilerParams(
            needs_layout_passes=False, use_tc_tiling_on_sc=False),
    )
    def run(pos_u, pos_v, un, vn, tin, tout, out,
            pu_v, pv_v, un_v, vn_v,
            ai0, ao0, vi0, vo0, ai1, ao1, vi1, vo1,
            uni0, uno0, vni0, vno0, uni1, uno1, vni1, vno1,
            stage, sem_a, sem_b):
        wid = lax.axis_index("s") * 2 + lax.axis_index("c")
        base = pl.multiple_of(wid * chunk, chunk)
        base5 = pl.multiple_of(wid * (chunk * _NS), chunk * _NS)
        pltpu.sync_copy(pos_u.at[pl.ds(base, chunk)], pu_v)
        pltpu.sync_copy(pos_v.at[pl.ds(base, chunk)], pv_v)
        pltpu.sync_copy(un.at[pl.ds(base5, chunk * _NS)], un_v)
        pltpu.sync_copy(vn.at[pl.ds(base5, chunk * _NS)], vn_v)

        bufs0 = (ai0, ao0, vi0, vo0, uni0, uno0, vni0, vno0)
        bufs1 = (ai1, ao1, vi1, vo1, uni1, uno1, vni1, vno1)

        def handles(g, bufs, sem):
            o = pl.multiple_of(g * _G, _G)
            o5 = pl.multiple_of(g * (_G * _NS), _G * _NS)
            ai, ao, vi, vo, uni, uno, vni, vno = bufs
            ipu = pu_v.at[pl.ds(o, _G)]
            ipv = pv_v.at[pl.ds(o, _G)]
            iun = un_v.at[pl.ds(o5, _G * _NS)]
            ivn = vn_v.at[pl.ds(o5, _G * _NS)]
            return [
                pltpu.make_async_copy(tin.at[ipu], ai, sem),
                pltpu.make_async_copy(tout.at[ipu], ao, sem),
                pltpu.make_async_copy(tin.at[ipv], vi, sem),
                pltpu.make_async_copy(tout.at[ipv], vo, sem),
                pltpu.make_async_copy(tin.at[iun], uni, sem),
                pltpu.make_async_copy(tout.at[iun], uno, sem),
                pltpu.make_async_copy(tin.at[ivn], vni, sem),
                pltpu.make_async_copy(tout.at[ivn], vno, sem),
            ]

        def fire(g, bufs, sem):
            for h in handles(g, bufs, sem):
                h.start()

        def drain(g, bufs, sem):
            for h in handles(g, bufs, sem):
                h.wait()

        iota = lax.iota(jnp.int32, 16)
        rn = [iota * _NS + k for k in range(_NS)]
        zeros = jnp.zeros((16,), jnp.float32)
        zcol = jnp.zeros((16,), jnp.int32)

        def halfdots(a, v, un, vn):
            def dbody(_, c):
                dots = list(c[:11])
                dc = c[11]
                a_x = plsc.load_gather(a, [iota, dc])
                v_x = plsc.load_gather(v, [iota, dc])
                dots[0] = dots[0] + a_x * v_x
                for k in range(_NS):
                    u_x = plsc.load_gather(un, [rn[k], dc])
                    w_x = plsc.load_gather(vn, [rn[k], dc])
                    dots[1 + k] = dots[1 + k] + u_x * v_x
                    dots[6 + k] = dots[6 + k] + w_x * a_x
                return (*dots, (dc + 1) & (_D - 1))

            out = lax.fori_loop(0, _D, dbody, (*((zeros,) * 11), iota),
                                unroll=2)
            return out[:11]

        def compute(bufs, accs):
            ai, ao, vi, vo, uni, uno, vni, vno = bufs
            din = halfdots(ai, vi, uni, vni)
            dout = halfdots(ao, vo, uno, vno)
            acc_pos = accs[0] + _log_sig(din[0]) + _log_sig(dout[0])
            news = [acc_pos]
            for k in range(_NS):
                news.append(accs[1 + k]
                            + _log_sig(-din[1 + k]) + _log_sig(-dout[1 + k])
                            + _log_sig(-din[6 + k]) + _log_sig(-dout[6 + k]))
            return tuple(news)

        fire(0, bufs0, sem_a)
        fire(1, bufs1, sem_b)

        def gbody(i, accs):
            g0 = 2 * i
            drain(g0, bufs0, sem_a)
            accs = compute(bufs0, accs)

            @pl.when(i < n_groups // 2 - 1)
            def _():
                fire(g0 + 2, bufs0, sem_a)

            drain(g0 + 1, bufs1, sem_b)
            accs = compute(bufs1, accs)

            @pl.when(i < n_groups // 2 - 1)
            def _():
                fire(g0 + 3, bufs1, sem_b)

            return accs

        accs = lax.fori_loop(0, n_groups // 2, gbody, (zeros,) * 6)
        for r in range(6):
            stage[pl.ds(r * 16, 16)] = accs[r]
        pltpu.sync_copy(stage, out.at[pl.ds(pl.multiple_of(wid * 96, 96), 96)])

    return run


def _noise_idx(key, n, ns, lo, hi):
    span = float(hi - lo - 1)
    return jnp.floor(jax.random.uniform(key, (n * ns,)) * span).astype(jnp.int32) + lo


def kernel(input_labels, out_labels, num_sampled, in_embed_weight, out_embed_weight):
    B, wp1 = out_labels.shape
    W = wp1 - 1
    BW = B * W
    types = input_labels[:, 0]
    ids = input_labels[:, 1]
    olf = out_labels.reshape(-1)
    j = jnp.arange(BW)
    jW = j // W
    jR = j % W

    pu, pv, un, vn = [], [], [], []
    n0 = None
    for tp in range(len(_EDGE_TYPES)):
        tu, tv, _ = _EDGE_TYPES[tp]
        sel = types == tp
        idxa = jnp.nonzero(sel, size=B, fill_value=0)[0]
        n = jnp.sum(sel.astype(jnp.int32))
        c_tp = lax.optimization_barrier(ids[idxa])
        jmod = lax.optimization_barrier(j % jnp.maximum(n, 1))
        pu.append(c_tp[jmod])
        r2 = lax.optimization_barrier(idxa[jW])
        vidx = lax.optimization_barrier(r2 * wp1 + 1 + jR)
        pv.append(olf[vidx])
        un.append(_noise_idx(jax.random.fold_in(jax.random.key(1), tp), BW, _NS,
                             _TYPE_OFFSET[tu], _TYPE_OFFSET[tu + 1]))
        vn.append(_noise_idx(jax.random.fold_in(jax.random.key(2), tp), BW, _NS,
                             _TYPE_OFFSET[tv], _TYPE_OFFSET[tv + 1]))
        if tp == 0:
            n0 = n

    shift = n0 * W

    def _rolled(x, s):
        size = x.shape[0]
        return lax.dynamic_slice(jnp.concatenate([x, x]), [size - s], [size])

    is0 = j < shift
    is0f = jnp.arange(BW * _NS) < shift * _NS
    pos_u = jnp.where(is0, pu[0], _rolled(pu[1], shift)).astype(jnp.int32)
    pos_v = jnp.where(is0, pv[0], _rolled(pv[1], shift)).astype(jnp.int32)
    unc = jnp.where(is0f, un[0], _rolled(un[1], shift * _NS))
    vnc = jnp.where(is0f, vn[0], _rolled(vn[1], shift * _NS))

    chunk = BW // _NW
    run = _make_sc_kernel(chunk, chunk // _G)
    parts = run(pos_u, pos_v, unc, vnc, in_embed_weight, out_embed_weight)
    parts = parts.reshape(_NW, 6, 16)
    pos_sum = parts[:, 0, :].sum()
    s = parts[:, 1:, :].sum(axis=(0, 2))
    colmask = jnp.arange(_NS) < num_sampled
    total = pos_sum + jnp.where(colmask, s, 0.0).sum() * 0.5
    return -total / BW


# speedup vs baseline: 3.6560x; 1.1038x over previous
import functools

import jax
import jax.numpy as jnp
from jax import lax
from jax.experimental import pallas as pl
from jax.experimental.pallas import tpu as pltpu
from jax.experimental.pallas import tpu_sc as plsc

_TYPE_OFFSET = [0, 500000, 1000000]
_EDGE_TYPES = [[0, 1, 0], [1, 0, 0]]
_NS = 5
_D = 64
_NW = 32
_G = 16
_B = 16384
_WP1 = 6
_W = _WP1 - 1
_BW = _B * _W
_CHUNK = _BW // _NW
_NG = _CHUNK // _G
_WIN = 520

_LN2 = 0.6931471805599453


def _log_sig(x):
    y = jnp.clip(x, -6.0, 6.0)
    z = 1.0 + jnp.exp(-y)
    b = lax.bitcast_convert_type(z, jnp.int32)
    e = jnp.right_shift(b, 23) - 127
    mb = jnp.bitwise_or(jnp.bitwise_and(b, 0x007FFFFF), 0x3F800000)
    m = lax.bitcast_convert_type(mb, jnp.float32)
    big = m > 1.5
    m = jnp.where(big, m * 0.5, m)
    e = (e + big.astype(jnp.int32)).astype(jnp.float32)
    t = (m - 1.0) / (m + 1.0)
    t2 = t * t
    p = 2.0 * t * (1.0 + t2 * (1.0 / 3.0 + t2 * (0.2 + t2 * (1.0 / 7.0 + t2 * (1.0 / 9.0)))))
    return -(e * _LN2 + p)


def _make_sc_kernel():
    mesh = plsc.VectorSubcoreMesh(core_axis_name="c", subcore_axis_name="s")
    scratch = (
        [pltpu.VMEM((2 * _B,), jnp.int32),
         pltpu.VMEM((_CHUNK,), jnp.int32),
         pltpu.VMEM((_CHUNK,), jnp.int32),
         pltpu.VMEM((_CHUNK * _NS,), jnp.int32),
         pltpu.VMEM((_CHUNK * _NS,), jnp.int32)]
        + [pltpu.VMEM((_G, _D), jnp.float32) for _ in range(8)]
        + [pltpu.VMEM((_G * _NS, _D), jnp.float32) for _ in range(8)]
        + [pltpu.VMEM((_G,), jnp.int32) for _ in range(2)]
        + [pltpu.VMEM((6 * 16,), jnp.float32),
           pltpu.SemaphoreType.DMA, pltpu.SemaphoreType.DMA]
    )

    @functools.partial(
        pl.kernel, mesh=mesh,
        out_type=jax.ShapeDtypeStruct((_NW * 96,), jnp.float32),
        scratch_types=scratch,
        compiler_params=pltpu.CompilerParams(
            needs_layout_passes=False, use_tc_tiling_on_sc=False),
    )
    def run(ccat, ucat, pos_v, un, vn, tin, tout, out,
            ccat_v, ucat_v, pv_v, un_v, vn_v,
            ai0, ao0, vi0, vo0, ai1, ao1, vi1, vo1,
            uni0, uno0, vni0, vno0, uni1, uno1, vni1, vno1,
            pu0, pu1,
            stage, sem_a, sem_b):
        wid = lax.axis_index("s") * 2 + lax.axis_index("c")
        base = pl.multiple_of(wid * _CHUNK, _CHUNK)
        base5 = pl.multiple_of(wid * (_CHUNK * _NS), _CHUNK * _NS)
        pltpu.sync_copy(ccat, ccat_v)
        pltpu.sync_copy(ucat.at[pl.ds(base, _CHUNK)], ucat_v)
        pltpu.sync_copy(pos_v.at[pl.ds(base, _CHUNK)], pv_v)
        pltpu.sync_copy(un.at[pl.ds(base5, _CHUNK * _NS)], un_v)
        pltpu.sync_copy(vn.at[pl.ds(base5, _CHUNK * _NS)], vn_v)
        iota = lax.iota(jnp.int32, 16)
        rn = [iota * _NS + k for k in range(_NS)]
        zeros = jnp.zeros((16,), jnp.float32)

        emb_bufs = ((ai0, ao0, vi0, vo0, uni0, uno0, vni0, vno0),
                    (ai1, ao1, vi1, vo1, uni1, uno1, vni1, vno1))
        emb_sems = (sem_a, sem_b)
        pu_bufs = (pu0, pu1)

        def ids_fire(g, slot):
            o = pl.multiple_of(g * _G, _G)
            ucvec = ucat_v[pl.ds(o, _G)]
            pu_bufs[slot][...] = plsc.load_gather(ccat_v, [ucvec])
            for h in emb_handles(g, slot):
                h.start()

        def emb_handles(g, slot):
            o = pl.multiple_of(g * _G, _G)
            o5 = pl.multiple_of(g * (_G * _NS), _G * _NS)
            ai, ao, vi, vo, uni, uno, vni, vno = emb_bufs[slot]
            sem = emb_sems[slot]
            ipu = pu_bufs[slot]
            ipv = pv_v.at[pl.ds(o, _G)]
            iun = un_v.at[pl.ds(o5, _G * _NS)]
            ivn = vn_v.at[pl.ds(o5, _G * _NS)]
            return [
                pltpu.make_async_copy(tin.at[ipu], ai, sem),
                pltpu.make_async_copy(tout.at[ipu], ao, sem),
                pltpu.make_async_copy(tin.at[ipv], vi, sem),
                pltpu.make_async_copy(tout.at[ipv], vo, sem),
                pltpu.make_async_copy(tin.at[iun], uni, sem),
                pltpu.make_async_copy(tout.at[iun], uno, sem),
                pltpu.make_async_copy(tin.at[ivn], vni, sem),
                pltpu.make_async_copy(tout.at[ivn], vno, sem),
            ]

        def drain_emb(g, slot):
            for h in emb_handles(g, slot):
                h.wait()

        def halfdots(a, v, un, vn):
            def dbody(_, c):
                dots = list(c[:11])
                dc = c[11]
                a_x = plsc.load_gather(a, [iota, dc])
                v_x = plsc.load_gather(v, [iota, dc])
                dots[0] = dots[0] + a_x * v_x
                for k in range(_NS):
                    u_x = plsc.load_gather(un, [rn[k], dc])
                    w_x = plsc.load_gather(vn, [rn[k], dc])
                    dots[1 + k] = dots[1 + k] + u_x * v_x
                    dots[6 + k] = dots[6 + k] + w_x * a_x
                return (*dots, (dc + 1) & (_D - 1))

            out_ = lax.fori_loop(0, _D, dbody, (*((zeros,) * 11), iota),
                                 unroll=2)
            return out_[:11]

        def compute(slot, accs):
            ai, ao, vi, vo, uni, uno, vni, vno = emb_bufs[slot]
            din = halfdots(ai, vi, uni, vni)
            dout = halfdots(ao, vo, uno, vno)
            acc_pos = accs[0] + _log_sig(din[0]) + _log_sig(dout[0])
            news = [acc_pos]
            for k in range(_NS):
                news.append(accs[1 + k]
                            + _log_sig(-din[1 + k]) + _log_sig(-dout[1 + k])
                            + _log_sig(-din[6 + k]) + _log_sig(-dout[6 + k]))
            return tuple(news)

        ids_fire(0, 0)
        ids_fire(1, 1)

        def gbody(i, accs):
            gb = 2 * i
            for sub in range(2):
                g = gb + sub
                slot = sub
                drain_emb(g, slot)
                accs = compute(slot, accs)

                @pl.when(g + 2 < _NG)
                def _():
                    ids_fire(g + 2, slot)
            return accs

        accs = lax.fori_loop(0, _NG // 2, gbody, (zeros,) * 6)
        for r in range(6):
            stage[pl.ds(r * 16, 16)] = accs[r]
        pltpu.sync_copy(stage, out.at[pl.ds(pl.multiple_of(wid * 96, 96), 96)])

    return run


def _noise_idx(key, n, ns, lo, hi):
    span = float(hi - lo - 1)
    return jnp.floor(jax.random.uniform(key, (n * ns,)) * span).astype(jnp.int32) + lo


def kernel(input_labels, out_labels, num_sampled, in_embed_weight, out_embed_weight):
    B, wp1 = out_labels.shape
    W = wp1 - 1
    BW = B * W
    types = input_labels[:, 0]
    ids = input_labels[:, 1]

    olf = out_labels.reshape(-1)
    j = jnp.arange(BW)
    jW = j // W
    jR = j % W

    cs, pv, un, vn, ns_ = [], [], [], [], []
    for tp in range(len(_EDGE_TYPES)):
        tu, tv, _ = _EDGE_TYPES[tp]
        sel = types == tp
        idxa = jnp.nonzero(sel, size=B, fill_value=0)[0].astype(jnp.int32)
        n = jnp.sum(sel.astype(jnp.int32))
        cs.append(lax.optimization_barrier(ids[idxa]).astype(jnp.int32))
        ns_.append(n)
        r2 = lax.optimization_barrier(idxa[jW])
        vidx = lax.optimization_barrier(r2 * wp1 + 1 + jR)
        pv.append(olf[vidx])
        un.append(_noise_idx(jax.random.fold_in(jax.random.key(1), tp), BW, _NS,
                             _TYPE_OFFSET[tu], _TYPE_OFFSET[tu + 1]))
        vn.append(_noise_idx(jax.random.fold_in(jax.random.key(2), tp), BW, _NS,
                             _TYPE_OFFSET[tv], _TYPE_OFFSET[tv + 1]))

    n0 = ns_[0]
    shift = n0 * W

    def _rolled(x, s):
        size = x.shape[0]
        return lax.dynamic_slice(jnp.concatenate([x, x]), [size - s], [size])

    is0 = j < shift
    is0f = jnp.arange(BW * _NS) < shift * _NS
    unc = jnp.where(is0f, un[0], _rolled(un[1], shift * _NS))
    vnc = jnp.where(is0f, vn[0], _rolled(vn[1], shift * _NS))
    pos_v = jnp.where(is0, pv[0], _rolled(pv[1], shift)).astype(jnp.int32)
    jm = jnp.where(is0, j, j - shift)
    ucat = jnp.where(is0, jm % jnp.maximum(n0, 1),
                     B + jm % jnp.maximum(ns_[1], 1)).astype(jnp.int32)

    ccat = jnp.concatenate(cs)

    run = _make_sc_kernel()
    parts = run(ccat, ucat, pos_v, unc, vnc,
                in_embed_weight, out_embed_weight)
    parts = parts.reshape(_NW, 6, 16)
    pos_sum = parts[:, 0, :].sum()
    s = parts[:, 1:, :].sum(axis=(0, 2))
    colmask = jnp.arange(_NS) < num_sampled
    total = pos_sum + jnp.where(colmask, s, 0.0).sum() * 0.5
    return -total / BW


# speedup vs baseline: 4.7169x; 1.2902x over previous
import functools

import jax
import jax.numpy as jnp
from jax import lax
from jax.experimental import pallas as pl
from jax.experimental.pallas import tpu as pltpu
from jax.experimental.pallas import tpu_sc as plsc

_TYPE_OFFSET = [0, 500000, 1000000]
_EDGE_TYPES = [[0, 1, 0], [1, 0, 0]]
_NS = 5
_D = 64
_NW = 32
_G = 16
_B = 16384
_WP1 = 6
_W = _WP1 - 1
_BW = _B * _W
_CHUNK = _BW // _NW
_NG = _CHUNK // _G
_WIN = 520

_LN2 = 0.6931471805599453


def _log_sig(x):
    y = jnp.clip(x, -6.0, 6.0)
    z = 1.0 + jnp.exp(-y)
    b = lax.bitcast_convert_type(z, jnp.int32)
    e = jnp.right_shift(b, 23) - 127
    mb = jnp.bitwise_or(jnp.bitwise_and(b, 0x007FFFFF), 0x3F800000)
    m = lax.bitcast_convert_type(mb, jnp.float32)
    big = m > 1.5
    m = jnp.where(big, m * 0.5, m)
    e = (e + big.astype(jnp.int32)).astype(jnp.float32)
    t = (m - 1.0) / (m + 1.0)
    t2 = t * t
    p = 2.0 * t * (1.0 + t2 * (1.0 / 3.0 + t2 * (0.2 + t2 * (1.0 / 7.0 + t2 * (1.0 / 9.0)))))
    return -(e * _LN2 + p)


def _make_sc_kernel():
    mesh = plsc.VectorSubcoreMesh(core_axis_name="c", subcore_axis_name="s")
    scratch = (
        [pltpu.VMEM((2 * _B,), jnp.int32),
         pltpu.VMEM((_CHUNK,), jnp.int32),
         pltpu.VMEM((_CHUNK,), jnp.int32),
         pltpu.VMEM((_CHUNK,), jnp.int32),
         pltpu.VMEM((_CHUNK * _NS,), jnp.int32),
         pltpu.VMEM((_CHUNK * _NS,), jnp.int32)]
        + [pltpu.VMEM((_G, _D), jnp.float32) for _ in range(8)]
        + [pltpu.VMEM((_G * _NS, _D), jnp.float32) for _ in range(8)]
        + [pltpu.VMEM((_G,), jnp.int32) for _ in range(2)]
        + [pltpu.VMEM((_G,), jnp.int32) for _ in range(2)]
        + [pltpu.VMEM((_G, 16), jnp.int32) for _ in range(4)]
        + [pltpu.VMEM((6 * 16,), jnp.float32),
           pltpu.SemaphoreType.DMA, pltpu.SemaphoreType.DMA,
           pltpu.SemaphoreType.DMA, pltpu.SemaphoreType.DMA,
           pltpu.SemaphoreType.DMA, pltpu.SemaphoreType.DMA]
    )

    @functools.partial(
        pl.kernel, mesh=mesh,
        out_type=jax.ShapeDtypeStruct((_NW * 96,), jnp.float32),
        scratch_types=scratch,
        compiler_params=pltpu.CompilerParams(
            needs_layout_passes=False, use_tc_tiling_on_sc=False),
    )
    def run(ccat, ucat, r2cat, colcat, olab, un, vn, tin, tout, out,
            ccat_v, ucat_v, r2_v, col_v, un_v, vn_v,
            ai0, ao0, vi0, vo0, ai1, ao1, vi1, vo1,
            uni0, uno0, vni0, vno0, uni1, uno1, vni1, vno1,
            pu0, pu1, pv0, pv1,
            vr0, vr1, vr2, vr3,
            stage, sem_a, sem_b, sv0, sv1, sv2, sv3):
        wid = lax.axis_index("s") * 2 + lax.axis_index("c")
        base = pl.multiple_of(wid * _CHUNK, _CHUNK)
        base5 = pl.multiple_of(wid * (_CHUNK * _NS), _CHUNK * _NS)
        pltpu.sync_copy(ccat, ccat_v)
        pltpu.sync_copy(ucat.at[pl.ds(base, _CHUNK)], ucat_v)
        pltpu.sync_copy(r2cat.at[pl.ds(base, _CHUNK)], r2_v)
        pltpu.sync_copy(colcat.at[pl.ds(base, _CHUNK)], col_v)
        pltpu.sync_copy(un.at[pl.ds(base5, _CHUNK * _NS)], un_v)
        pltpu.sync_copy(vn.at[pl.ds(base5, _CHUNK * _NS)], vn_v)
        iota = lax.iota(jnp.int32, 16)
        rn = [iota * _NS + k for k in range(_NS)]
        zeros = jnp.zeros((16,), jnp.float32)

        emb_bufs = ((ai0, ao0, vi0, vo0, uni0, uno0, vni0, vno0),
                    (ai1, ao1, vi1, vo1, uni1, uno1, vni1, vno1))
        emb_sems = (sem_a, sem_b)
        pu_bufs = (pu0, pu1)
        pv_bufs = (pv0, pv1)
        vrow_bufs = (vr0, vr1, vr2, vr3)
        vrow_sems = (sv0, sv1, sv2, sv3)

        def vrow_handle(g, ring):
            o = pl.multiple_of(g * _G, _G)
            return pltpu.make_async_copy(olab.at[r2_v.at[pl.ds(o, _G)]],
                                         vrow_bufs[ring], vrow_sems[ring])

        def ids_fire(g, slot, ring):
            o = pl.multiple_of(g * _G, _G)
            ucvec = ucat_v[pl.ds(o, _G)]
            pu_bufs[slot][...] = plsc.load_gather(ccat_v, [ucvec])
            vrow_handle(g, ring).wait()
            col = col_v[pl.ds(o, _G)]
            pv_bufs[slot][...] = plsc.load_gather(vrow_bufs[ring], [iota, col])
            for h in emb_handles(g, slot):
                h.start()

        def emb_handles(g, slot):
            o = pl.multiple_of(g * _G, _G)
            o5 = pl.multiple_of(g * (_G * _NS), _G * _NS)
            ai, ao, vi, vo, uni, uno, vni, vno = emb_bufs[slot]
            sem = emb_sems[slot]
            ipu = pu_bufs[slot]
            ipv = pv_bufs[slot]
            iun = un_v.at[pl.ds(o5, _G * _NS)]
            ivn = vn_v.at[pl.ds(o5, _G * _NS)]
            return [
                pltpu.make_async_copy(tin.at[ipu], ai, sem),
                pltpu.make_async_copy(tout.at[ipu], ao, sem),
                pltpu.make_async_copy(tin.at[ipv], vi, sem),
                pltpu.make_async_copy(tout.at[ipv], vo, sem),
                pltpu.make_async_copy(tin.at[iun], uni, sem),
                pltpu.make_async_copy(tout.at[iun], uno, sem),
                pltpu.make_async_copy(tin.at[ivn], vni, sem),
                pltpu.make_async_copy(tout.at[ivn], vno, sem),
            ]

        def drain_emb(g, slot):
            for h in emb_handles(g, slot):
                h.wait()

        def halfdots(a, v, un, vn):
            def dbody(_, c):
                dots = list(c[:11])
                dc = c[11]
                a_x = plsc.load_gather(a, [iota, dc])
                v_x = plsc.load_gather(v, [iota, dc])
                dots[0] = dots[0] + a_x * v_x
                for k in range(_NS):
                    u_x = plsc.load_gather(un, [rn[k], dc])
                    w_x = plsc.load_gather(vn, [rn[k], dc])
                    dots[1 + k] = dots[1 + k] + u_x * v_x
                    dots[6 + k] = dots[6 + k] + w_x * a_x
                return (*dots, (dc + 1) & (_D - 1))

            out_ = lax.fori_loop(0, _D, dbody, (*((zeros,) * 11), iota),
                                 unroll=2)
            return out_[:11]

        def compute(slot, accs):
            ai, ao, vi, vo, uni, uno, vni, vno = emb_bufs[slot]
            din = halfdots(ai, vi, uni, vni)
            dout = halfdots(ao, vo, uno, vno)
            acc_pos = accs[0] + _log_sig(din[0]) + _log_sig(dout[0])
            news = [acc_pos]
            for k in range(_NS):
                news.append(accs[1 + k]
                            + _log_sig(-din[1 + k]) + _log_sig(-dout[1 + k])
                            + _log_sig(-din[6 + k]) + _log_sig(-dout[6 + k]))
            return tuple(news)

        for g in range(4):
            vrow_handle(g, g).start()
        ids_fire(0, 0, 0)
        ids_fire(1, 1, 1)

        def gbody(i, accs):
            gb = 4 * i
            for sub in range(4):
                g = gb + sub
                slot = sub % 2
                drain_emb(g, slot)
                accs = compute(slot, accs)

                @pl.when(g + 2 < _NG)
                def _():
                    ids_fire(g + 2, slot, (sub + 2) % 4)

                @pl.when(g + 4 < _NG)
                def _():
                    vrow_handle(g + 4, sub).start()
            return accs

        accs = lax.fori_loop(0, _NG // 4, gbody, (zeros,) * 6)
        for r in range(6):
            stage[pl.ds(r * 16, 16)] = accs[r]
        pltpu.sync_copy(stage, out.at[pl.ds(pl.multiple_of(wid * 96, 96), 96)])

    return run


def _noise_idx(key, n, ns, lo, hi):
    span = float(hi - lo - 1)
    return jnp.floor(jax.random.uniform(key, (n * ns,)) * span).astype(jnp.int32) + lo


def kernel(input_labels, out_labels, num_sampled, in_embed_weight, out_embed_weight):
    B, wp1 = out_labels.shape
    W = wp1 - 1
    BW = B * W
    types = input_labels[:, 0]
    ids = input_labels[:, 1]

    j = jnp.arange(BW)
    jW = j // W

    cs, r2s, un, vn, ns_ = [], [], [], [], []
    for tp in range(len(_EDGE_TYPES)):
        tu, tv, _ = _EDGE_TYPES[tp]
        sel = types == tp
        idxa = jnp.nonzero(sel, size=B, fill_value=0)[0].astype(jnp.int32)
        n = jnp.sum(sel.astype(jnp.int32))
        cs.append(lax.optimization_barrier(ids[idxa]).astype(jnp.int32))
        ns_.append(n)
        r2s.append(lax.optimization_barrier(idxa[jW]))
        un.append(_noise_idx(jax.random.fold_in(jax.random.key(1), tp), BW, _NS,
                             _TYPE_OFFSET[tu], _TYPE_OFFSET[tu + 1]))
        vn.append(_noise_idx(jax.random.fold_in(jax.random.key(2), tp), BW, _NS,
                             _TYPE_OFFSET[tv], _TYPE_OFFSET[tv + 1]))

    n0 = ns_[0]
    shift = n0 * W

    def _rolled(x, s):
        size = x.shape[0]
        return lax.dynamic_slice(jnp.concatenate([x, x]), [size - s], [size])

    is0 = j < shift
    is0f = jnp.arange(BW * _NS) < shift * _NS
    unc = jnp.where(is0f, un[0], _rolled(un[1], shift * _NS))
    vnc = jnp.where(is0f, vn[0], _rolled(vn[1], shift * _NS))
    r2cat = jnp.where(is0, r2s[0], _rolled(r2s[1], shift)).astype(jnp.int32)
    jm = jnp.where(is0, j, j - shift)
    ucat = jnp.where(is0, jm % jnp.maximum(n0, 1),
                     B + jm % jnp.maximum(ns_[1], 1)).astype(jnp.int32)
    colcat = (1 + jm % W).astype(jnp.int32)

    ccat = jnp.concatenate(cs)
    olab16 = jnp.pad(out_labels.astype(jnp.int32), ((0, 0), (0, 16 - wp1)))

    run = _make_sc_kernel()
    parts = run(ccat, ucat, r2cat, colcat, olab16, unc, vnc,
                in_embed_weight, out_embed_weight)
    parts = parts.reshape(_NW, 6, 16)
    pos_sum = parts[:, 0, :].sum()
    s = parts[:, 1:, :].sum(axis=(0, 2))
    colmask = jnp.arange(_NS) < num_sampled
    total = pos_sum + jnp.where(colmask, s, 0.0).sum() * 0.5
    return -total / BW


# speedup vs baseline: 7.6290x; 1.6174x over previous
import functools

import jax
import jax.numpy as jnp
from jax import lax
from jax.experimental import pallas as pl
from jax.experimental.pallas import tpu as pltpu
from jax.experimental.pallas import tpu_sc as plsc

_TYPE_OFFSET = [0, 500000, 1000000]
_EDGE_TYPES = [[0, 1, 0], [1, 0, 0]]
_NS = 5
_D = 64
_NW = 32
_G = 16
_B = 16384
_WP1 = 6
_W = _WP1 - 1
_BW = _B * _W
_CHUNK = _BW // _NW
_NG = _CHUNK // _G
_WIN = 520

_LN2 = 0.6931471805599453


def _log_sig(x):
    y = jnp.clip(x, -6.0, 6.0)
    z = 1.0 + jnp.exp(-y)
    b = lax.bitcast_convert_type(z, jnp.int32)
    e = jnp.right_shift(b, 23) - 127
    mb = jnp.bitwise_or(jnp.bitwise_and(b, 0x007FFFFF), 0x3F800000)
    m = lax.bitcast_convert_type(mb, jnp.float32)
    big = m > 1.5
    m = jnp.where(big, m * 0.5, m)
    e = (e + big.astype(jnp.int32)).astype(jnp.float32)
    t = (m - 1.0) / (m + 1.0)
    t2 = t * t
    p = 2.0 * t * (1.0 + t2 * (1.0 / 3.0 + t2 * (0.2 + t2 * (1.0 / 7.0 + t2 * (1.0 / 9.0)))))
    return -(e * _LN2 + p)


def _make_sc_kernel():
    mesh = plsc.VectorSubcoreMesh(core_axis_name="c", subcore_axis_name="s")
    scratch = (
        [pltpu.VMEM((2 * _B,), jnp.int32),
         pltpu.VMEM((_CHUNK,), jnp.int32),
         pltpu.VMEM((_CHUNK,), jnp.int32),
         pltpu.VMEM((_CHUNK,), jnp.int32),
         pltpu.VMEM((_CHUNK * _NS,), jnp.int32),
         pltpu.VMEM((_CHUNK * _NS,), jnp.int32)]
        + [pltpu.VMEM((_G, _D), jnp.float32) for _ in range(8)]
        + [pltpu.VMEM((_G * _NS, _D), jnp.float32) for _ in range(8)]
        + [pltpu.VMEM((_G,), jnp.int32) for _ in range(2)]
        + [pltpu.VMEM((_G,), jnp.int32) for _ in range(2)]
        + [pltpu.VMEM((_G, 16), jnp.int32) for _ in range(4)]
        + [pltpu.VMEM((6 * 16,), jnp.float32),
           pltpu.SemaphoreType.DMA, pltpu.SemaphoreType.DMA,
           pltpu.SemaphoreType.DMA, pltpu.SemaphoreType.DMA,
           pltpu.SemaphoreType.DMA, pltpu.SemaphoreType.DMA]
    )

    @functools.partial(
        pl.kernel, mesh=mesh,
        out_type=jax.ShapeDtypeStruct((_NW * 96,), jnp.float32),
        scratch_types=scratch,
        compiler_params=pltpu.CompilerParams(
            needs_layout_passes=False, use_tc_tiling_on_sc=False),
    )
    def run(ccat, ucat, r2cat, colcat, olab, un, vn, tin, tout, out,
            ccat_v, ucat_v, r2_v, col_v, un_v, vn_v,
            ai0, ao0, vi0, vo0, ai1, ao1, vi1, vo1,
            uni0, uno0, vni0, vno0, uni1, uno1, vni1, vno1,
            pu0, pu1, pv0, pv1,
            vr0, vr1, vr2, vr3,
            stage, sem_a, sem_b, sv0, sv1, sv2, sv3):
        wid = lax.axis_index("s") * 2 + lax.axis_index("c")
        base = pl.multiple_of(wid * _CHUNK, _CHUNK)
        base5 = pl.multiple_of(wid * (_CHUNK * _NS), _CHUNK * _NS)
        pltpu.sync_copy(ccat, ccat_v)
        pltpu.sync_copy(ucat.at[pl.ds(base, _CHUNK)], ucat_v)
        pltpu.sync_copy(r2cat.at[pl.ds(base, _CHUNK)], r2_v)
        pltpu.sync_copy(colcat.at[pl.ds(base, _CHUNK)], col_v)
        pltpu.sync_copy(un.at[pl.ds(base5, _CHUNK * _NS)], un_v)
        pltpu.sync_copy(vn.at[pl.ds(base5, _CHUNK * _NS)], vn_v)
        iota = lax.iota(jnp.int32, 16)
        rn = [iota * _NS + k for k in range(_NS)]
        zeros = jnp.zeros((16,), jnp.float32)

        emb_bufs = ((ai0, ao0, vi0, vo0, uni0, uno0, vni0, vno0),
                    (ai1, ao1, vi1, vo1, uni1, uno1, vni1, vno1))
        emb_sems = (sem_a, sem_b)
        pu_bufs = (pu0, pu1)
        pv_bufs = (pv0, pv1)
        vrow_bufs = (vr0, vr1, vr2, vr3)
        vrow_sems = (sv0, sv1, sv2, sv3)

        def vrow_handle(g, ring):
            o = pl.multiple_of(g * _G, _G)
            return pltpu.make_async_copy(olab.at[r2_v.at[pl.ds(o, _G)]],
                                         vrow_bufs[ring], vrow_sems[ring])

        def ids_fire(g, slot, ring):
            o = pl.multiple_of(g * _G, _G)
            ucvec = ucat_v[pl.ds(o, _G)]
            pu_bufs[slot][...] = plsc.load_gather(ccat_v, [ucvec])
            vrow_handle(g, ring).wait()
            col = col_v[pl.ds(o, _G)]
            pv_bufs[slot][...] = plsc.load_gather(vrow_bufs[ring], [iota, col])
            for h in emb_handles(g, slot):
                h.start()

        def emb_handles(g, slot):
            o = pl.multiple_of(g * _G, _G)
            o5 = pl.multiple_of(g * (_G * _NS), _G * _NS)
            ai, ao, vi, vo, uni, uno, vni, vno = emb_bufs[slot]
            sem = emb_sems[slot]
            ipu = pu_bufs[slot]
            ipv = pv_bufs[slot]
            iun = un_v.at[pl.ds(o5, _G * _NS)]
            ivn = vn_v.at[pl.ds(o5, _G * _NS)]
            return [
                pltpu.make_async_copy(tin.at[ipu], ai, sem),
                pltpu.make_async_copy(tout.at[ipu], ao, sem),
                pltpu.make_async_copy(tin.at[ipv], vi, sem),
                pltpu.make_async_copy(tout.at[ipv], vo, sem),
                pltpu.make_async_copy(tin.at[iun], uni, sem),
                pltpu.make_async_copy(tout.at[iun], uno, sem),
                pltpu.make_async_copy(tin.at[ivn], vni, sem),
                pltpu.make_async_copy(tout.at[ivn], vno, sem),
            ]

        def drain_emb(g, slot):
            for h in emb_handles(g, slot):
                h.wait()

        def halfdots(a, v, un, vn):
            def dbody(_, c):
                dots = list(c[:11])
                dc = c[11]
                a_x = plsc.load_gather(a, [iota, dc])
                v_x = plsc.load_gather(v, [iota, dc])
                dots[0] = dots[0] + a_x * v_x
                for k in range(_NS):
                    u_x = plsc.load_gather(un, [rn[k], dc])
                    w_x = plsc.load_gather(vn, [rn[k], dc])
                    dots[1 + k] = dots[1 + k] + u_x * v_x
                    dots[6 + k] = dots[6 + k] + w_x * a_x
                return (*dots, (dc + 1) & (_D - 1))

            out_ = lax.fori_loop(0, _D, dbody, (*((zeros,) * 11), iota),
                                 unroll=2)
            return out_[:11]

        def compute(slot, accs):
            ai, ao, vi, vo, uni, uno, vni, vno = emb_bufs[slot]
            din = halfdots(ai, vi, uni, vni)
            dout = halfdots(ao, vo, uno, vno)
            acc_pos = accs[0] + _log_sig(din[0]) + _log_sig(dout[0])
            news = [acc_pos]
            for k in range(_NS):
                news.append(accs[1 + k]
                            + _log_sig(-din[1 + k]) + _log_sig(-dout[1 + k])
                            + _log_sig(-din[6 + k]) + _log_sig(-dout[6 + k]))
            return tuple(news)

        for g in range(4):
            vrow_handle(g, g).start()
        ids_fire(0, 0, 0)
        ids_fire(1, 1, 1)

        def gbody(i, accs):
            gb = 4 * i
            for sub in range(4):
                g = gb + sub
                slot = sub % 2
                drain_emb(g, slot)
                accs = compute(slot, accs)

                @pl.when(g + 2 < _NG)
                def _():
                    ids_fire(g + 2, slot, (sub + 2) % 4)

                @pl.when(g + 4 < _NG)
                def _():
                    vrow_handle(g + 4, sub).start()
            return accs

        accs = lax.fori_loop(0, _NG // 4, gbody, (zeros,) * 6)
        for r in range(6):
            stage[pl.ds(r * 16, 16)] = accs[r]
        pltpu.sync_copy(stage, out.at[pl.ds(pl.multiple_of(wid * 96, 96), 96)])

    return run


def _noise_idx(key, n, ns, lo, hi):
    span = float(hi - lo - 1)
    return jnp.floor(jax.random.uniform(key, (n * ns,)) * span).astype(jnp.int32) + lo


def kernel(input_labels, out_labels, num_sampled, in_embed_weight, out_embed_weight):
    B, wp1 = out_labels.shape
    W = wp1 - 1
    BW = B * W
    types = input_labels[:, 0]
    ids = input_labels[:, 1]

    j = jnp.arange(BW)
    jW = j // W

    cs, r2s, un, vn, ns_ = [], [], [], [], []
    for tp in range(len(_EDGE_TYPES)):
        tu, tv, _ = _EDGE_TYPES[tp]
        sel = types == tp
        idxa = jnp.nonzero(sel, size=B, fill_value=0)[0].astype(jnp.int32)
        n = jnp.sum(sel.astype(jnp.int32))
        cs.append(lax.optimization_barrier(ids[idxa]).astype(jnp.int32))
        ns_.append(n)
        r2s.append(jnp.repeat(idxa, W))
        un.append(_noise_idx(jax.random.fold_in(jax.random.key(1), tp), BW, _NS,
                             _TYPE_OFFSET[tu], _TYPE_OFFSET[tu + 1]))
        vn.append(_noise_idx(jax.random.fold_in(jax.random.key(2), tp), BW, _NS,
                             _TYPE_OFFSET[tv], _TYPE_OFFSET[tv + 1]))

    n0 = ns_[0]
    shift = n0 * W

    def _rolled(x, s):
        size = x.shape[0]
        return lax.dynamic_slice(jnp.concatenate([x, x]), [size - s], [size])

    is0 = j < shift
    is0f = jnp.arange(BW * _NS) < shift * _NS
    unc = jnp.where(is0f, un[0], _rolled(un[1], shift * _NS))
    vnc = jnp.where(is0f, vn[0], _rolled(vn[1], shift * _NS))
    r2cat = jnp.where(is0, r2s[0], _rolled(r2s[1], shift)).astype(jnp.int32)
    jm = jnp.where(is0, j, j - shift)
    ucat = jnp.where(is0, jm % jnp.maximum(n0, 1),
                     B + jm % jnp.maximum(ns_[1], 1)).astype(jnp.int32)
    colcat = (1 + jm % W).astype(jnp.int32)

    ccat = jnp.concatenate(cs)
    olab16 = jnp.pad(out_labels.astype(jnp.int32), ((0, 0), (0, 16 - wp1)))

    run = _make_sc_kernel()
    parts = run(ccat, ucat, r2cat, colcat, olab16, unc, vnc,
                in_embed_weight, out_embed_weight)
    parts = parts.reshape(_NW, 6, 16)
    pos_sum = parts[:, 0, :].sum()
    s = parts[:, 1:, :].sum(axis=(0, 2))
    colmask = jnp.arange(_NS) < num_sampled
    total = pos_sum + jnp.where(colmask, s, 0.0).sum() * 0.5
    return -total / BW
